# baseline (device time: 83505 ns/iter reference)
import os

import jax
import jax.numpy as jnp
from jax import lax
from jax.experimental import pallas as pl
from jax.experimental.pallas import tpu as pltpu

T = 2048
V_LOCAL = 16384
D = 1024

_MODE = os.environ.get("KMODE", "full")


def kernel(ids, E):
    def body(ids_ref, e_ref, out_ref, ptok_ref, prow_ref,
             local_sem, send_sem, recv_sem):
        my_x = lax.axis_index("x")
        my_y = lax.axis_index("y")
        my_z = lax.axis_index("z")
        partner = (1 - my_x, my_y, my_z)

        barrier = pltpu.get_barrier_semaphore()
        pl.semaphore_signal(
            barrier, inc=1, device_id=partner,
            device_id_type=pl.DeviceIdType.MESH,
        )
        pl.semaphore_wait(barrier, 1)

        base = my_x * V_LOCAL
        UNROLL = 8

        def issue_rdma(j, n_mine):
            for u in range(UNROLL):
                i = j * UNROLL + u
                tok = ids_ref[i]
                row = tok - base
                owned = (row >= 0) & (row < V_LOCAL)

                @pl.when(owned)
                def _():
                    if _MODE != "nordma":
                        pltpu.make_async_remote_copy(
                            src_ref=e_ref.at[pl.ds(row, 1), :],
                            dst_ref=out_ref.at[pl.ds(i, 1), :],
                            send_sem=send_sem,
                            recv_sem=recv_sem,
                            device_id=partner,
                            device_id_type=pl.DeviceIdType.MESH,
                        ).start()
                    ptok_ref[n_mine] = i
                    prow_ref[n_mine] = row

                n_mine = n_mine + jnp.where(owned, 1, 0)
            return n_mine

        n_mine = lax.fori_loop(0, T // UNROLL, issue_rdma, jnp.int32(0))
        n_theirs = T - n_mine

        if _MODE != "nolocal":
            def issue_local(j, _):
                for u in range(UNROLL):
                    k = j * UNROLL + u

                    @pl.when(k < n_mine)
                    def _():
                        pltpu.make_async_copy(
                            e_ref.at[pl.ds(prow_ref[k], 1), :],
                            out_ref.at[pl.ds(ptok_ref[k], 1), :],
                            local_sem,
                        ).start()
                return 0

            lax.fori_loop(0, T // UNROLL, issue_local, 0)

        def drain(sem, count, is_recv, is_remote):
            for k in reversed(range(T.bit_length())):
                w = 1 << k
                if w > T:
                    continue

                @pl.when((count & w) != 0)
                def _():
                    if is_remote:
                        d = pltpu.make_async_remote_copy(
                            src_ref=e_ref.at[pl.ds(0, w), :],
                            dst_ref=out_ref.at[pl.ds(0, w), :],
                            send_sem=send_sem,
                            recv_sem=recv_sem,
                            device_id=partner,
                            device_id_type=pl.DeviceIdType.MESH,
                        )
                        d.wait_recv() if is_recv else d.wait_send()
                    else:
                        pltpu.make_async_copy(
                            e_ref.at[pl.ds(0, w), :],
                            out_ref.at[pl.ds(0, w), :],
                            sem,
                        ).wait()

        if _MODE != "nolocal":
            drain(local_sem, n_mine, False, False)
        if _MODE != "nordma":
            drain(send_sem, n_mine, False, True)
            drain(recv_sem, n_theirs, True, True)

    return pl.pallas_call(
        body,
        out_shape=jax.ShapeDtypeStruct((T, D), jnp.float32),
        in_specs=[
            pl.BlockSpec(memory_space=pltpu.SMEM),
            pl.BlockSpec(memory_space=pl.ANY),
        ],
        out_specs=pl.BlockSpec(memory_space=pltpu.VMEM),
        scratch_shapes=[
            pltpu.SMEM((T,), jnp.int32),
            pltpu.SMEM((T,), jnp.int32),
            pltpu.SemaphoreType.DMA,
            pltpu.SemaphoreType.DMA,
            pltpu.SemaphoreType.DMA,
        ],
        compiler_params=pltpu.CompilerParams(collective_id=0),
    )(ids, E)


# device time: 68861 ns/iter; 1.2127x vs baseline; 1.2127x over previous
import os

import jax
import jax.numpy as jnp
from jax import lax
from jax.experimental import pallas as pl
from jax.experimental.pallas import tpu as pltpu

T = 2048
V_LOCAL = 16384
SHIFT = 14
D = 1024

_MODE = os.environ.get("KMODE", "full")


def kernel(ids, E):
    my_x = lax.axis_index("x")
    base = my_x * V_LOCAL
    n_mine = jnp.sum(
        ((ids - base) >> SHIFT) == 0, dtype=jnp.int32
    ).reshape((1,))

    def body(ids_ref, nm_ref, e_ref, out_ref, local_sem, send_sem, recv_sem):
        my_x = lax.axis_index("x")
        my_y = lax.axis_index("y")
        my_z = lax.axis_index("z")
        partner = (1 - my_x, my_y, my_z)
        base = my_x * V_LOCAL

        barrier = pltpu.get_barrier_semaphore()
        pl.semaphore_signal(
            barrier, inc=1, device_id=partner,
            device_id_type=pl.DeviceIdType.MESH,
        )
        pl.semaphore_wait(barrier, 1)

        UNROLL = 16

        def issue(j, carry):
            for u in range(UNROLL):
                i = j * UNROLL + u
                row = ids_ref[i] - base
                owned = (row >> SHIFT) == 0

                @pl.when(owned)
                def _():
                    if _MODE != "nolocal":
                        pltpu.make_async_copy(
                            e_ref.at[pl.ds(row, 1), :],
                            out_ref.at[pl.ds(i, 1), :],
                            local_sem,
                        ).start()
                    if _MODE != "nordma":
                        pltpu.make_async_remote_copy(
                            src_ref=e_ref.at[pl.ds(row, 1), :],
                            dst_ref=out_ref.at[pl.ds(i, 1), :],
                            send_sem=send_sem,
                            recv_sem=recv_sem,
                            device_id=partner,
                            device_id_type=pl.DeviceIdType.MESH,
                        ).start()
            return carry

        lax.fori_loop(0, T // UNROLL, issue, 0)
        n_mine = nm_ref[0]
        n_theirs = T - n_mine

        def drain(sem, count, is_recv, is_remote):
            for k in reversed(range(T.bit_length())):
                w = 1 << k
                if w > T:
                    continue

                @pl.when((count & w) != 0)
                def _():
                    if is_remote:
                        d = pltpu.make_async_remote_copy(
                            src_ref=e_ref.at[pl.ds(0, w), :],
                            dst_ref=out_ref.at[pl.ds(0, w), :],
                            send_sem=send_sem,
                            recv_sem=recv_sem,
                            device_id=partner,
                            device_id_type=pl.DeviceIdType.MESH,
                        )
                        d.wait_recv() if is_recv else d.wait_send()
                    else:
                        pltpu.make_async_copy(
                            e_ref.at[pl.ds(0, w), :],
                            out_ref.at[pl.ds(0, w), :],
                            sem,
                        ).wait()

        if _MODE != "nolocal":
            drain(local_sem, n_mine, False, False)
        if _MODE != "nordma":
            drain(send_sem, n_mine, False, True)
            drain(recv_sem, n_theirs, True, True)

    return pl.pallas_call(
        body,
        out_shape=jax.ShapeDtypeStruct((T, D), jnp.float32),
        in_specs=[
            pl.BlockSpec(memory_space=pltpu.SMEM),
            pl.BlockSpec(memory_space=pltpu.SMEM),
            pl.BlockSpec(memory_space=pl.ANY),
        ],
        out_specs=pl.BlockSpec(memory_space=pltpu.VMEM),
        scratch_shapes=[
            pltpu.SemaphoreType.DMA,
            pltpu.SemaphoreType.DMA,
            pltpu.SemaphoreType.DMA,
        ],
        compiler_params=pltpu.CompilerParams(collective_id=0),
    )(ids, n_mine, E)


# device time: 68581 ns/iter; 1.2176x vs baseline; 1.0041x over previous
import os

import jax
import jax.numpy as jnp
from jax import lax
from jax.experimental import pallas as pl
from jax.experimental.pallas import tpu as pltpu

T = 2048
V_LOCAL = 16384
SHIFT = 14
D = 1024

_MODE = os.environ.get("KMODE", "full")


def kernel(ids, E):
    my_x = lax.axis_index("x")
    base = my_x * V_LOCAL
    n_mine = jnp.sum(
        ((ids - base) >> SHIFT) == 0, dtype=jnp.int32
    ).reshape((1,))

    def body(ids_ref, nm_ref, e_ref, out_ref, local_sem, send_sem, recv_sem):
        my_x = lax.axis_index("x")
        my_y = lax.axis_index("y")
        my_z = lax.axis_index("z")
        base = my_x * V_LOCAL
        partner = (1 - my_x) * 16 + my_y * 4 + my_z

        barrier = pltpu.get_barrier_semaphore()
        pl.semaphore_signal(
            barrier, inc=1, device_id=partner,
            device_id_type=pl.DeviceIdType.LOGICAL,
        )
        pl.semaphore_wait(barrier, 1)

        UNROLL = 32

        def issue(j, carry):
            for u in range(UNROLL):
                i = j * UNROLL + u
                row = ids_ref[i] - base
                owned = (row >> SHIFT) == 0

                @pl.when(owned)
                def _():
                    if _MODE != "nolocal":
                        pltpu.make_async_copy(
                            e_ref.at[pl.ds(row, 1), :],
                            out_ref.at[pl.ds(i, 1), :],
                            local_sem,
                        ).start()
                    if _MODE != "nordma":
                        pltpu.make_async_remote_copy(
                            src_ref=e_ref.at[pl.ds(row, 1), :],
                            dst_ref=out_ref.at[pl.ds(i, 1), :],
                            send_sem=send_sem,
                            recv_sem=recv_sem,
                            device_id=partner,
                            device_id_type=pl.DeviceIdType.LOGICAL,
                        ).start()
            return carry

        lax.fori_loop(0, T // UNROLL, issue, 0)
        n_mine = nm_ref[0]
        n_theirs = T - n_mine

        def drain(sem, count, is_recv, is_remote):
            for k in reversed(range(T.bit_length())):
                w = 1 << k
                if w > T:
                    continue

                @pl.when((count & w) != 0)
                def _():
                    if is_remote:
                        d = pltpu.make_async_remote_copy(
                            src_ref=e_ref.at[pl.ds(0, w), :],
                            dst_ref=out_ref.at[pl.ds(0, w), :],
                            send_sem=send_sem,
                            recv_sem=recv_sem,
                            device_id=partner,
                            device_id_type=pl.DeviceIdType.LOGICAL,
                        )
                        d.wait_recv() if is_recv else d.wait_send()
                    else:
                        pltpu.make_async_copy(
                            e_ref.at[pl.ds(0, w), :],
                            out_ref.at[pl.ds(0, w), :],
                            sem,
                        ).wait()

        if _MODE != "nolocal":
            drain(local_sem, n_mine, False, False)
        if _MODE != "nordma":
            drain(send_sem, n_mine, False, True)
            drain(recv_sem, n_theirs, True, True)

    return pl.pallas_call(
        body,
        out_shape=jax.ShapeDtypeStruct((T, D), jnp.float32),
        in_specs=[
            pl.BlockSpec(memory_space=pltpu.SMEM),
            pl.BlockSpec(memory_space=pltpu.SMEM),
            pl.BlockSpec(memory_space=pl.ANY),
        ],
        out_specs=pl.BlockSpec(memory_space=pl.ANY),
        scratch_shapes=[
            pltpu.SemaphoreType.DMA,
            pltpu.SemaphoreType.DMA,
            pltpu.SemaphoreType.DMA,
        ],
        compiler_params=pltpu.CompilerParams(collective_id=0),
    )(ids, n_mine, E)
